# baseline (speedup 1.0000x reference)
import functools

import jax
import jax.numpy as jnp
import numpy as np
from jax import lax
from jax.experimental import pallas as pl
from jax.experimental.pallas import tpu as pltpu
from jax.experimental.pallas import tpu_sc as plsc

_N = 10000
_E = 320000
_D = 128

_NC = 2
_NS = 16
_NW = _NC * _NS
_CHUNK = 48
_GC = 8
_G0 = 42
_G1 = 12
_GMAX = 42
_EPW0 = _G0 * _GC * _CHUNK
_EPW1 = _G1 * _GC * _CHUNK
_E0 = 256000
_NPAD = 10112
_ROWS_PER_TILE = _NPAD // _NS

_DELTA = np.empty(_D, np.int32)
for _k in range(_D // 32):
    for _t in range(16):
        _DELTA[32 * _k + _t] = 32 * _k + 2 * _t
        _DELTA[32 * _k + 16 + _t] = 32 * _k + 2 * _t + 1
_CPERM = np.argsort(_DELTA)


def _mm_bias_body(x_ref, w_ref, b_ref, o_ref):
    o_ref[...] = (
        jnp.dot(x_ref[...], w_ref[...], preferred_element_type=jnp.float32)
        + b_ref[...]
    )


def _mm_bias(x, w, b):
    return pl.pallas_call(
        _mm_bias_body,
        out_shape=jax.ShapeDtypeStruct(
            (x.shape[0], w.shape[1]), jnp.float32),
    )(x, w, b.reshape(1, -1))


def _combine_relu_mm_body(p0_ref, p1_ref, w_ref, b_ref, o_ref):
    h = jnp.maximum(p0_ref[...] + p1_ref[...], 0.0)
    o_ref[...] = (
        jnp.dot(h, w_ref[...], preferred_element_type=jnp.float32)
        + b_ref[...]
    )


def _combine_relu_mm(p0, p1, w, b):
    return pl.pallas_call(
        _combine_relu_mm_body,
        out_shape=jax.ShapeDtypeStruct(
            (p0.shape[0], w.shape[1]), jnp.float32),
    )(p0, p1, w, b.reshape(1, -1))


def _add_body(p0_ref, p1_ref, o_ref):
    o_ref[...] = p0_ref[...] + p1_ref[...]


def _combine_add(p0, p1):
    return pl.pallas_call(
        _add_body,
        out_shape=jax.ShapeDtypeStruct(p0.shape, jnp.float32),
    )(p0, p1)


def _bcast_lane(vec16, k):
    idx = jnp.full((16, 1), k, jnp.int32)
    dnums = lax.GatherDimensionNumbers(
        offset_dims=(), collapsed_slice_dims=(0,), start_index_map=(0,))
    return lax.gather(
        vec16, idx, dnums, (1,),
        mode=lax.GatherScatterMode.PROMISE_IN_BOUNDS)


def _sc_body(sup_hbm, src_hbm, dst_hbm, w_hbm, out_hbm,
             src_v, dst_v, w_v,
             in0, in1, in2, in3, o0, o1, acc,
             g0, g1, g2, g3, s0, s1, esem):
    c = lax.axis_index("c")
    s = lax.axis_index("s")
    glim = jnp.where(c == 0, _G0, _G1)
    ins = [in0, in1, in2, in3]
    outs = [o0, o1]
    gsems = [g0, g1, g2, g3]
    ssems = [s0, s1]

    def _zrow(r, carry):
        for cc in range(_D // 16):
            o0[r, pl.ds(cc * 16, 16)] = jnp.zeros((16,), jnp.float32)
        return carry
    lax.fori_loop(0, _CHUNK, _zrow, 0, unroll=4)
    base = s * _ROWS_PER_TILE
    nz = _ROWS_PER_TILE // _CHUNK
    for z in range(nz):
        pltpu.sync_copy(o0, acc.at[pl.ds(base + z * _CHUNK, _CHUNK)])
    rem = _ROWS_PER_TILE - nz * _CHUNK
    if rem:
        pltpu.sync_copy(
            o0.at[pl.ds(0, rem)],
            acc.at[pl.ds(base + nz * _CHUNK, rem)])
    plsc.subcore_barrier()

    def _edges_for(h, buf, sync):
        cp = pltpu.sync_copy if sync else (
            lambda a, b: pltpu.async_copy(a, b, esem))
        cp(src_hbm.at[c, s, h], src_v.at[buf])
        cp(dst_hbm.at[c, s, h], dst_v.at[buf])
        cp(w_hbm.at[c, s, h], w_v.at[buf])

    def _wait_edges(h, buf):
        for ref_h, ref_v in ((src_hbm, src_v), (dst_hbm, dst_v),
                             (w_hbm, w_v)):
            pltpu.make_async_copy(
                ref_h.at[c, s, h], ref_v.at[buf], esem).wait()

    def _start_gather(hbuf, row, b):
        pltpu.async_copy(
            sup_hbm.at[src_v.at[hbuf, row]], ins[b], gsems[b])

    def _wait_gather(hbuf, row, b):
        pltpu.make_async_copy(
            sup_hbm.at[src_v.at[hbuf, row]], ins[b], gsems[b]).wait()

    def _start_scatter(hbuf, row, ob):
        pltpu.async_copy(
            outs[ob], acc.at[dst_v.at[hbuf, row]], ssems[ob], add=True)

    def _wait_scatter(hbuf, row, ob):
        pltpu.make_async_copy(
            outs[ob], acc.at[dst_v.at[hbuf, row]], ssems[ob]).wait()

    def _scale(hbuf, row, b, ob):
        def _g16(g, gcarry):
            wv = w_v[hbuf, row, pl.ds(g * 16, 16)]

            def _lane(k, lcarry):
                r = g * 16 + k
                wvec = _bcast_lane(wv, k)
                for cc in range(_D // 16):
                    sl = pl.ds(cc * 16, 16)
                    outs[ob][r, sl] = ins[b][r, sl] * wvec
                return lcarry

            lax.fori_loop(0, 16, _lane, 0, unroll=4)
            return gcarry
        lax.fori_loop(0, _CHUNK // 16, _g16, 0)

    _edges_for(0, 0, True)
    _edges_for(1, 1, False)
    _start_gather(0, 0, 0)
    _start_gather(0, 1, 1)

    def _group(h, carry):
        hb = lax.rem(h, 2)
        hb1 = lax.rem(h + 1, 2)
        for cig in range(_GC):
            b = cig % 4
            pb = (cig + 2) % 4
            ob = cig % 2
            if cig == 2:
                @pl.when(jnp.logical_and(h >= 1, h < glim - 1))
                def _():
                    _edges_for(h + 1, hb1, False)
            if cig == 6:
                @pl.when(h < glim - 1)
                def _():
                    _wait_edges(h + 1, hb1)
            if cig < _GC - 2:
                _start_gather(hb, cig + 2, pb)
            else:
                @pl.when(h < glim - 1)
                def _():
                    _start_gather(hb1, cig - _GC + 2, pb)
            _wait_gather(hb, cig, b)
            if cig < 2:
                @pl.when(h >= 1)
                def _():
                    _wait_scatter(hb1, cig + _GC - 2, ob)
            else:
                _wait_scatter(hb, cig - 2, ob)
            _scale(hb, cig, b, ob)
            _start_scatter(hb, cig, ob)
        return carry

    lax.fori_loop(0, glim, _group, 0)
    _wait_scatter(1, _GC - 2, 0)
    _wait_scatter(1, _GC - 1, 1)

    plsc.subcore_barrier()

    pltpu.sync_copy(
        acc.at[pl.ds(base, _ROWS_PER_TILE)],
        out_hbm.at[c, pl.ds(base, _ROWS_PER_TILE)],
    )


_sc_scatter = functools.partial(
    pl.kernel,
    mesh=plsc.VectorSubcoreMesh(
        core_axis_name="c", subcore_axis_name="s",
        num_cores=_NC, num_subcores=_NS),
    out_type=jax.ShapeDtypeStruct((_NC, _NPAD, _D), jnp.float32),
    scratch_types=(
        [
            pltpu.VMEM((2, _GC, _CHUNK), jnp.int32),
            pltpu.VMEM((2, _GC, _CHUNK), jnp.int32),
            pltpu.VMEM((2, _GC, _CHUNK), jnp.float32),
        ]
        + [pltpu.VMEM((_CHUNK, _D), jnp.float32)] * 4
        + [pltpu.VMEM((_CHUNK, _D), jnp.float32)] * 2
        + [pltpu.VMEM_SHARED((_NPAD, _D), jnp.float32)]
        + [pltpu.SemaphoreType.DMA] * 7
    ),
)(_sc_body)


def _layout_edges(x):
    n0 = _E0 // _NS
    x0 = jnp.pad(x[:_E0].reshape(_NS, n0), ((0, 0), (0, _EPW0 - n0)))
    x0 = x0.reshape(_NS, _GMAX, _GC, _CHUNK)
    n1 = (_E - _E0) // _NS
    x1 = jnp.pad(x[_E0:].reshape(_NS, n1), ((0, 0), (0, _EPW1 - n1)))
    x1 = x1.reshape(_NS, _G1, _GC, _CHUNK)
    x1 = jnp.pad(x1, ((0, 0), (0, _GMAX - _G1), (0, 0), (0, 0)))
    return jnp.stack([x0, x1])


@jax.jit
def kernel(inp, edge_index, edge_weight, W1, b1, W2, b2):
    srcp = _layout_edges(edge_index[0])
    dstp = _layout_edges(edge_index[1])
    wp = _layout_edges(edge_weight)

    s1 = _mm_bias(inp, W1, b1)
    p = _sc_scatter(s1, srcp, dstp, wp)
    s2 = _combine_relu_mm(p[0, :_N], p[1, :_N], W2, b2)
    q = _sc_scatter(s2, srcp, dstp, wp)
    return _combine_add(q[0, :_N], q[1, :_N])


# speedup vs baseline: 1.6358x; 1.6358x over previous
import functools

import jax
import jax.numpy as jnp
import numpy as np
from jax import lax
from jax.experimental import pallas as pl
from jax.experimental.pallas import tpu as pltpu
from jax.experimental.pallas import tpu_sc as plsc

_N = 10000
_E = 320000
_D = 128

_NC = 2
_NS = 16
_NW = _NC * _NS
_CHUNK = 128
_GC = 8
_G0 = 10
_G1 = 10
_GMAX = 10
_EPW0 = _G0 * _GC * _CHUNK
_EPW1 = _G1 * _GC * _CHUNK
_E0 = _E // 2
_NPAD = 10112
_ROWS_PER_TILE = _NPAD // _NS

_DELTA = np.empty(_D, np.int32)
for _k in range(_D // 32):
    for _t in range(16):
        _DELTA[32 * _k + _t] = 32 * _k + 2 * _t
        _DELTA[32 * _k + 16 + _t] = 32 * _k + 2 * _t + 1
_CPERM = np.argsort(_DELTA)


def _mm_bias_body(x_ref, w_ref, b_ref, o_ref):
    o_ref[...] = (
        jnp.dot(x_ref[...], w_ref[...], preferred_element_type=jnp.float32)
        + b_ref[...]
    )


def _mm_bias(x, w, b):
    return pl.pallas_call(
        _mm_bias_body,
        out_shape=jax.ShapeDtypeStruct(
            (x.shape[0], w.shape[1]), jnp.float32),
    )(x, w, b.reshape(1, -1))


def _combine_relu_mm_body(p0_ref, p1_ref, w_ref, b_ref, o_ref):
    h = jnp.maximum(p0_ref[...] + p1_ref[...], 0.0)
    o_ref[...] = (
        jnp.dot(h, w_ref[...], preferred_element_type=jnp.float32)
        + b_ref[...]
    )


def _combine_relu_mm(p0, p1, w, b):
    return pl.pallas_call(
        _combine_relu_mm_body,
        out_shape=jax.ShapeDtypeStruct(
            (p0.shape[0], w.shape[1]), jnp.float32),
    )(p0, p1, w, b.reshape(1, -1))


def _add_body(p0_ref, p1_ref, o_ref):
    o_ref[...] = p0_ref[...] + p1_ref[...]


def _combine_add(p0, p1):
    return pl.pallas_call(
        _add_body,
        out_shape=jax.ShapeDtypeStruct(p0.shape, jnp.float32),
    )(p0, p1)


def _bcast_lane(vec16, k):
    idx = jnp.full((16, 1), k, jnp.int32)
    dnums = lax.GatherDimensionNumbers(
        offset_dims=(), collapsed_slice_dims=(0,), start_index_map=(0,))
    return lax.gather(
        vec16, idx, dnums, (1,),
        mode=lax.GatherScatterMode.PROMISE_IN_BOUNDS)


def _sc_body(sup_hbm, src_hbm, dst_hbm, w_hbm, out_hbm,
             src_v, dst_v, w_v,
             in0, in1, acc,
             g0, g1, esem):
    c = lax.axis_index("c")
    s = lax.axis_index("s")
    glim = jnp.where(c == 0, _G0, _G1)
    ins = [in0, in1]
    gsems = [g0, g1]

    def _zrow(r, carry):
        for cc in range(_D // 16):
            in0[r, pl.ds(cc * 16, 16)] = jnp.zeros((16,), jnp.float32)
        return carry
    lax.fori_loop(0, _CHUNK, _zrow, 0, unroll=4)
    base = s * _ROWS_PER_TILE
    nz = _ROWS_PER_TILE // _CHUNK
    for z in range(nz):
        pltpu.sync_copy(in0, acc.at[pl.ds(base + z * _CHUNK, _CHUNK)])
    rem = _ROWS_PER_TILE - nz * _CHUNK
    if rem:
        pltpu.sync_copy(
            in0.at[pl.ds(0, rem)],
            acc.at[pl.ds(base + nz * _CHUNK, rem)])
    plsc.subcore_barrier()

    def _edges_for(h, buf, sync):
        cp = pltpu.sync_copy if sync else (
            lambda a, b: pltpu.async_copy(a, b, esem))
        cp(src_hbm.at[c, s, h], src_v.at[buf])
        cp(dst_hbm.at[c, s, h], dst_v.at[buf])
        cp(w_hbm.at[c, s, h], w_v.at[buf])

    def _wait_edges(h, buf):
        for ref_h, ref_v in ((src_hbm, src_v), (dst_hbm, dst_v),
                             (w_hbm, w_v)):
            pltpu.make_async_copy(
                ref_h.at[c, s, h], ref_v.at[buf], esem).wait()

    def _start_gather(hbuf, row, b):
        pltpu.async_copy(
            sup_hbm.at[src_v.at[hbuf, row]], ins[b], gsems[b])

    def _wait_gather(hbuf, row, b):
        pltpu.make_async_copy(
            sup_hbm.at[src_v.at[hbuf, row]], ins[b], gsems[b]).wait()

    def _scale(hbuf, row, b):
        def _g16(g, gcarry):
            wv = w_v[hbuf, row, pl.ds(g * 16, 16)]
            for k in range(16):
                r = g * 16 + k
                wvec = _bcast_lane(wv, k)
                for cc in range(_D // 16):
                    sl = pl.ds(cc * 16, 16)
                    ins[b][r, sl] = ins[b][r, sl] * wvec
            return gcarry
        lax.fori_loop(0, _CHUNK // 16, _g16, 0)

    _edges_for(0, 0, True)
    _edges_for(1, 1, False)
    _start_gather(0, 0, 0)

    def _group(h, carry):
        hb = lax.rem(h, 2)
        hb1 = lax.rem(h + 1, 2)
        for cig in range(_GC):
            b = cig % 2
            if cig == 2:
                @pl.when(jnp.logical_and(h >= 1, h < glim - 1))
                def _():
                    _edges_for(h + 1, hb1, False)
            if cig == 6:
                @pl.when(h < glim - 1)
                def _():
                    _wait_edges(h + 1, hb1)
            _wait_gather(hb, cig, b)
            if cig < _GC - 1:
                _start_gather(hb, cig + 1, 1 - b)
            else:
                @pl.when(h < glim - 1)
                def _():
                    _start_gather(hb1, 0, 1 - b)
            _scale(hb, cig, b)
            pltpu.sync_copy(
                ins[b], acc.at[dst_v.at[hb, cig]], add=True)
        return carry

    lax.fori_loop(0, glim, _group, 0)

    plsc.subcore_barrier()

    pltpu.sync_copy(
        acc.at[pl.ds(base, _ROWS_PER_TILE)],
        out_hbm.at[c, pl.ds(base, _ROWS_PER_TILE)],
    )


_sc_scatter = functools.partial(
    pl.kernel,
    mesh=plsc.VectorSubcoreMesh(
        core_axis_name="c", subcore_axis_name="s",
        num_cores=_NC, num_subcores=_NS),
    out_type=jax.ShapeDtypeStruct((_NC, _NPAD, _D), jnp.float32),
    scratch_types=(
        [
            pltpu.VMEM((2, _GC, _CHUNK), jnp.int32),
            pltpu.VMEM((2, _GC, _CHUNK), jnp.int32),
            pltpu.VMEM((2, _GC, _CHUNK), jnp.float32),
        ]
        + [pltpu.VMEM((_CHUNK, _D), jnp.float32)] * 2
        + [pltpu.VMEM_SHARED((_NPAD, _D), jnp.float32)]
        + [pltpu.SemaphoreType.DMA] * 3
    ),
)(_sc_body)


def _layout_edges(x):
    n0 = _E0 // _NS
    x0 = jnp.pad(x[:_E0].reshape(_NS, n0), ((0, 0), (0, _EPW0 - n0)))
    x0 = x0.reshape(_NS, _GMAX, _GC, _CHUNK)
    n1 = (_E - _E0) // _NS
    x1 = jnp.pad(x[_E0:].reshape(_NS, n1), ((0, 0), (0, _EPW1 - n1)))
    x1 = x1.reshape(_NS, _G1, _GC, _CHUNK)
    x1 = jnp.pad(x1, ((0, 0), (0, _GMAX - _G1), (0, 0), (0, 0)))
    return jnp.stack([x0, x1])


@jax.jit
def kernel(inp, edge_index, edge_weight, W1, b1, W2, b2):
    srcp = _layout_edges(edge_index[0])
    dstp = _layout_edges(edge_index[1])
    wp = _layout_edges(edge_weight)

    s1 = _mm_bias(inp, W1, b1)
    p = _sc_scatter(s1, srcp, dstp, wp)
    s2 = _combine_relu_mm(p[0, :_N], p[1, :_N], W2, b2)
    q = _sc_scatter(s2, srcp, dstp, wp)
    return _combine_add(q[0, :_N], q[1, :_N])


# speedup vs baseline: 1.6610x; 1.0154x over previous
import functools

import jax
import jax.numpy as jnp
import numpy as np
from jax import lax
from jax.experimental import pallas as pl
from jax.experimental.pallas import tpu as pltpu
from jax.experimental.pallas import tpu_sc as plsc

_N = 10000
_E = 320000
_D = 128

_NC = 2
_NS = 16
_NW = _NC * _NS
_CHUNK = 128
_GC = 8
_G0 = 10
_G1 = 10
_GMAX = 10
_EPW0 = _G0 * _GC * _CHUNK
_EPW1 = _G1 * _GC * _CHUNK
_E0 = _E // 2
_NPAD = 10112
_ROWS_PER_TILE = _NPAD // _NS

_DELTA = np.empty(_D, np.int32)
for _k in range(_D // 32):
    for _t in range(16):
        _DELTA[32 * _k + _t] = 32 * _k + 2 * _t
        _DELTA[32 * _k + 16 + _t] = 32 * _k + 2 * _t + 1
_CPERM = np.argsort(_DELTA)


def _mm_bias_body(x_ref, w_ref, b_ref, o_ref):
    o_ref[...] = (
        jnp.dot(x_ref[...], w_ref[...], preferred_element_type=jnp.float32)
        + b_ref[...]
    )


def _mm_bias(x, w, b):
    return pl.pallas_call(
        _mm_bias_body,
        out_shape=jax.ShapeDtypeStruct(
            (x.shape[0], w.shape[1]), jnp.float32),
    )(x, w, b.reshape(1, -1))


def _combine_relu_mm_body(p_ref, w_ref, b_ref, o_ref):
    h = jnp.maximum(p_ref[0, :_N, :] + p_ref[1, :_N, :], 0.0)
    o_ref[...] = (
        jnp.dot(h, w_ref[...], preferred_element_type=jnp.float32)
        + b_ref[...]
    )


def _combine_relu_mm(p, w, b):
    return pl.pallas_call(
        _combine_relu_mm_body,
        out_shape=jax.ShapeDtypeStruct((_N, w.shape[1]), jnp.float32),
    )(p, w, b.reshape(1, -1))


def _add_body(q_ref, o_ref):
    o_ref[...] = q_ref[0, :_N, :] + q_ref[1, :_N, :]


def _combine_add(q):
    return pl.pallas_call(
        _add_body,
        out_shape=jax.ShapeDtypeStruct((_N, _D), jnp.float32),
    )(q)


def _bcast_lane(vec16, k):
    idx = jnp.full((16, 1), k, jnp.int32)
    dnums = lax.GatherDimensionNumbers(
        offset_dims=(), collapsed_slice_dims=(0,), start_index_map=(0,))
    return lax.gather(
        vec16, idx, dnums, (1,),
        mode=lax.GatherScatterMode.PROMISE_IN_BOUNDS)


def _sc_body(sup_hbm, src_hbm, dst_hbm, w_hbm, out_hbm,
             src_v, dst_v, w_v,
             in0, in1, acc,
             g0, g1, esem):
    c = lax.axis_index("c")
    s = lax.axis_index("s")
    glim = _G0
    ins = [in0, in1]
    gsems = [g0, g1]

    def _zrow(r, carry):
        for cc in range(_D // 16):
            in0[r, pl.ds(cc * 16, 16)] = jnp.zeros((16,), jnp.float32)
        return carry
    lax.fori_loop(0, _CHUNK, _zrow, 0, unroll=4)
    base = s * _ROWS_PER_TILE
    nz = _ROWS_PER_TILE // _CHUNK
    for z in range(nz):
        pltpu.sync_copy(in0, acc.at[pl.ds(base + z * _CHUNK, _CHUNK)])
    rem = _ROWS_PER_TILE - nz * _CHUNK
    if rem:
        pltpu.sync_copy(
            in0.at[pl.ds(0, rem)],
            acc.at[pl.ds(base + nz * _CHUNK, rem)])
    plsc.subcore_barrier()

    def _edges_for(h, buf, sync):
        cp = pltpu.sync_copy if sync else (
            lambda a, b: pltpu.async_copy(a, b, esem))
        cp(src_hbm.at[c, s, h], src_v.at[buf])
        cp(dst_hbm.at[c, s, h], dst_v.at[buf])
        cp(w_hbm.at[c, s, h], w_v.at[buf])

    def _wait_edges(h, buf):
        for ref_h, ref_v in ((src_hbm, src_v), (dst_hbm, dst_v),
                             (w_hbm, w_v)):
            pltpu.make_async_copy(
                ref_h.at[c, s, h], ref_v.at[buf], esem).wait()

    def _start_gather(hbuf, row, b):
        pltpu.async_copy(
            sup_hbm.at[src_v.at[hbuf, row]], ins[b], gsems[b])

    def _wait_gather(hbuf, row, b):
        pltpu.make_async_copy(
            sup_hbm.at[src_v.at[hbuf, row]], ins[b], gsems[b]).wait()

    def _scale(hbuf, row, b):
        def _g16(g, gcarry):
            wv = w_v[hbuf, row, pl.ds(g * 16, 16)]
            for k in range(16):
                r = g * 16 + k
                wvec = _bcast_lane(wv, k)
                for cc in range(_D // 16):
                    sl = pl.ds(cc * 16, 16)
                    ins[b][r, sl] = ins[b][r, sl] * wvec
            return gcarry
        lax.fori_loop(0, _CHUNK // 16, _g16, 0)

    _edges_for(0, 0, True)
    _edges_for(1, 1, False)
    _start_gather(0, 0, 0)

    def _group(h, carry):
        hb = lax.rem(h, 2)
        hb1 = lax.rem(h + 1, 2)
        for cig in range(_GC):
            b = cig % 2
            if cig == 2:
                @pl.when(jnp.logical_and(h >= 1, h < glim - 1))
                def _():
                    _edges_for(h + 1, hb1, False)
            if cig == 6:
                @pl.when(h < glim - 1)
                def _():
                    _wait_edges(h + 1, hb1)
            _wait_gather(hb, cig, b)
            if cig < _GC - 1:
                _start_gather(hb, cig + 1, 1 - b)
            else:
                @pl.when(h < glim - 1)
                def _():
                    _start_gather(hb1, 0, 1 - b)
            _scale(hb, cig, b)
            pltpu.sync_copy(
                ins[b], acc.at[dst_v.at[hb, cig]], add=True)
        return carry

    lax.fori_loop(0, glim, _group, 0)

    plsc.subcore_barrier()

    pltpu.sync_copy(
        acc.at[pl.ds(base, _ROWS_PER_TILE)],
        out_hbm.at[c, pl.ds(base, _ROWS_PER_TILE)],
    )


_sc_scatter = functools.partial(
    pl.kernel,
    mesh=plsc.VectorSubcoreMesh(
        core_axis_name="c", subcore_axis_name="s",
        num_cores=_NC, num_subcores=_NS),
    out_type=jax.ShapeDtypeStruct((_NC, _NPAD, _D), jnp.float32),
    scratch_types=(
        [
            pltpu.VMEM((2, _GC, _CHUNK), jnp.int32),
            pltpu.VMEM((2, _GC, _CHUNK), jnp.int32),
            pltpu.VMEM((2, _GC, _CHUNK), jnp.float32),
        ]
        + [pltpu.VMEM((_CHUNK, _D), jnp.float32)] * 2
        + [pltpu.VMEM_SHARED((_NPAD, _D), jnp.float32)]
        + [pltpu.SemaphoreType.DMA] * 3
    ),
)(_sc_body)


def _layout_edges(x):
    n0 = _E0 // _NS
    x0 = jnp.pad(x[:_E0].reshape(_NS, n0), ((0, 0), (0, _EPW0 - n0)))
    x0 = x0.reshape(_NS, _GMAX, _GC, _CHUNK)
    n1 = (_E - _E0) // _NS
    x1 = jnp.pad(x[_E0:].reshape(_NS, n1), ((0, 0), (0, _EPW1 - n1)))
    x1 = x1.reshape(_NS, _G1, _GC, _CHUNK)
    x1 = jnp.pad(x1, ((0, 0), (0, _GMAX - _G1), (0, 0), (0, 0)))
    return jnp.stack([x0, x1])


@jax.jit
def kernel(inp, edge_index, edge_weight, W1, b1, W2, b2):
    srcp = _layout_edges(edge_index[0])
    dstp = _layout_edges(edge_index[1])
    wp = _layout_edges(edge_weight)

    s1 = _mm_bias(inp, W1, b1)
    p = _sc_scatter(s1, srcp, dstp, wp)
    s2 = _combine_relu_mm(p, W2, b2)
    q = _sc_scatter(s2, srcp, dstp, wp)
    return _combine_add(q)


# speedup vs baseline: 1.6855x; 1.0147x over previous
import functools

import jax
import jax.numpy as jnp
import numpy as np
from jax import lax
from jax.experimental import pallas as pl
from jax.experimental.pallas import tpu as pltpu
from jax.experimental.pallas import tpu_sc as plsc

_N = 10000
_E = 320000
_D = 128

_NC = 2
_NS = 16
_NW = _NC * _NS
_CHUNK = 80
_GC = 8
_G0 = 16
_G1 = 16
_GMAX = 16
_EPW0 = _G0 * _GC * _CHUNK
_EPW1 = _G1 * _GC * _CHUNK
_E0 = _E // 2
_NPAD = 10112
_ROWS_PER_TILE = _NPAD // _NS

_DELTA = np.empty(_D, np.int32)
for _k in range(_D // 32):
    for _t in range(16):
        _DELTA[32 * _k + _t] = 32 * _k + 2 * _t
        _DELTA[32 * _k + 16 + _t] = 32 * _k + 2 * _t + 1
_CPERM = np.argsort(_DELTA)


def _mm_bias_body(x_ref, w_ref, b_ref, o_ref):
    o_ref[...] = (
        jnp.dot(x_ref[...], w_ref[...], preferred_element_type=jnp.float32)
        + b_ref[...]
    )


def _mm_bias(x, w, b):
    return pl.pallas_call(
        _mm_bias_body,
        out_shape=jax.ShapeDtypeStruct(
            (x.shape[0], w.shape[1]), jnp.float32),
    )(x, w, b.reshape(1, -1))


def _combine_relu_mm_body(p_ref, w_ref, b_ref, o_ref):
    h = jnp.maximum(p_ref[0, :_N, :] + p_ref[1, :_N, :], 0.0)
    o_ref[...] = (
        jnp.dot(h, w_ref[...], preferred_element_type=jnp.float32)
        + b_ref[...]
    )


def _combine_relu_mm(p, w, b):
    return pl.pallas_call(
        _combine_relu_mm_body,
        out_shape=jax.ShapeDtypeStruct((_N, w.shape[1]), jnp.float32),
    )(p, w, b.reshape(1, -1))


def _add_body(q_ref, o_ref):
    o_ref[...] = q_ref[0, :_N, :] + q_ref[1, :_N, :]


def _combine_add(q):
    return pl.pallas_call(
        _add_body,
        out_shape=jax.ShapeDtypeStruct((_N, _D), jnp.float32),
    )(q)


def _bcast_lane(vec16, k):
    idx = jnp.full((16, 1), k, jnp.int32)
    dnums = lax.GatherDimensionNumbers(
        offset_dims=(), collapsed_slice_dims=(0,), start_index_map=(0,))
    return lax.gather(
        vec16, idx, dnums, (1,),
        mode=lax.GatherScatterMode.PROMISE_IN_BOUNDS)


def _sc_body(sup_hbm, src_hbm, dst_hbm, w_hbm, out_hbm,
             src_v, dst_v, w_v,
             in0, in1, in2, in3, acc,
             g0, g1, g2, g3, s0, s1, s2, s3, esem):
    c = lax.axis_index("c")
    s = lax.axis_index("s")
    glim = _G0
    ins = [in0, in1, in2, in3]
    gsems = [g0, g1, g2, g3]
    ssems = [s0, s1, s2, s3]

    def _zrow(r, carry):
        for cc in range(_D // 16):
            in0[r, pl.ds(cc * 16, 16)] = jnp.zeros((16,), jnp.float32)
        return carry
    lax.fori_loop(0, _CHUNK, _zrow, 0, unroll=4)
    base = s * _ROWS_PER_TILE
    nz = _ROWS_PER_TILE // _CHUNK
    for z in range(nz):
        pltpu.sync_copy(in0, acc.at[pl.ds(base + z * _CHUNK, _CHUNK)])
    rem = _ROWS_PER_TILE - nz * _CHUNK
    if rem:
        pltpu.sync_copy(
            in0.at[pl.ds(0, rem)],
            acc.at[pl.ds(base + nz * _CHUNK, rem)])
    plsc.subcore_barrier()

    def _edges_for(h, buf, sync):
        cp = pltpu.sync_copy if sync else (
            lambda a, b: pltpu.async_copy(a, b, esem))
        cp(src_hbm.at[c, s, h], src_v.at[buf])
        cp(dst_hbm.at[c, s, h], dst_v.at[buf])
        cp(w_hbm.at[c, s, h], w_v.at[buf])

    def _wait_edges(h, buf):
        for ref_h, ref_v in ((src_hbm, src_v), (dst_hbm, dst_v),
                             (w_hbm, w_v)):
            pltpu.make_async_copy(
                ref_h.at[c, s, h], ref_v.at[buf], esem).wait()

    def _start_gather(hbuf, row, b):
        pltpu.async_copy(
            sup_hbm.at[src_v.at[hbuf, row]], ins[b], gsems[b])

    def _wait_gather(hbuf, row, b):
        pltpu.make_async_copy(
            sup_hbm.at[src_v.at[hbuf, row]], ins[b], gsems[b]).wait()

    def _scale(hbuf, row, b):
        def _g16(g, gcarry):
            wv = w_v[hbuf, row, pl.ds(g * 16, 16)]
            for k in range(16):
                r = g * 16 + k
                wvec = _bcast_lane(wv, k)
                for cc in range(_D // 16):
                    sl = pl.ds(cc * 16, 16)
                    ins[b][r, sl] = ins[b][r, sl] * wvec
            return gcarry
        lax.fori_loop(0, _CHUNK // 16, _g16, 0)

    def _start_scatter(hbuf, row, b):
        pltpu.async_copy(
            ins[b], acc.at[dst_v.at[hbuf, row]], ssems[b], add=True)

    def _wait_scatter(hbuf, row, b):
        pltpu.make_async_copy(
            ins[b], acc.at[dst_v.at[hbuf, row]], ssems[b]).wait()

    _edges_for(0, 0, True)
    _edges_for(1, 1, False)
    _start_gather(0, 0, 0)
    _start_gather(0, 1, 1)

    def _group(h, carry):
        hb = lax.rem(h, 2)
        hb1 = lax.rem(h + 1, 2)
        for cig in range(_GC):
            b = cig % 4
            pb = (cig + 2) % 4
            if cig == 2:
                @pl.when(jnp.logical_and(h >= 1, h < glim - 1))
                def _():
                    _edges_for(h + 1, hb1, False)
            if cig == 6:
                @pl.when(h < glim - 1)
                def _():
                    _wait_edges(h + 1, hb1)
            _wait_gather(hb, cig, b)
            if cig < 2:
                @pl.when(h >= 1)
                def _():
                    _wait_scatter(hb1, cig + _GC - 2, pb)
            else:
                _wait_scatter(hb, cig - 2, pb)
            if cig < _GC - 2:
                _start_gather(hb, cig + 2, pb)
            else:
                @pl.when(h < glim - 1)
                def _():
                    _start_gather(hb1, cig - _GC + 2, pb)
            _scale(hb, cig, b)
            _start_scatter(hb, cig, b)
        return carry

    lax.fori_loop(0, glim, _group, 0)
    _wait_scatter(1, _GC - 2, 2)
    _wait_scatter(1, _GC - 1, 3)

    plsc.subcore_barrier()

    pltpu.sync_copy(
        acc.at[pl.ds(base, _ROWS_PER_TILE)],
        out_hbm.at[c, pl.ds(base, _ROWS_PER_TILE)],
    )


_sc_scatter = functools.partial(
    pl.kernel,
    mesh=plsc.VectorSubcoreMesh(
        core_axis_name="c", subcore_axis_name="s",
        num_cores=_NC, num_subcores=_NS),
    out_type=jax.ShapeDtypeStruct((_NC, _NPAD, _D), jnp.float32),
    scratch_types=(
        [
            pltpu.VMEM((2, _GC, _CHUNK), jnp.int32),
            pltpu.VMEM((2, _GC, _CHUNK), jnp.int32),
            pltpu.VMEM((2, _GC, _CHUNK), jnp.float32),
        ]
        + [pltpu.VMEM((_CHUNK, _D), jnp.float32)] * 4
        + [pltpu.VMEM_SHARED((_NPAD, _D), jnp.float32)]
        + [pltpu.SemaphoreType.DMA] * 9
    ),
)(_sc_body)


def _layout_edges(x):
    n0 = _E0 // _NS
    x0 = jnp.pad(x[:_E0].reshape(_NS, n0), ((0, 0), (0, _EPW0 - n0)))
    x0 = x0.reshape(_NS, _GMAX, _GC, _CHUNK)
    n1 = (_E - _E0) // _NS
    x1 = jnp.pad(x[_E0:].reshape(_NS, n1), ((0, 0), (0, _EPW1 - n1)))
    x1 = x1.reshape(_NS, _G1, _GC, _CHUNK)
    x1 = jnp.pad(x1, ((0, 0), (0, _GMAX - _G1), (0, 0), (0, 0)))
    return jnp.stack([x0, x1])


@jax.jit
def kernel(inp, edge_index, edge_weight, W1, b1, W2, b2):
    srcp = _layout_edges(edge_index[0])
    dstp = _layout_edges(edge_index[1])
    wp = _layout_edges(edge_weight)

    s1 = _mm_bias(inp, W1, b1)
    p = _sc_scatter(s1, srcp, dstp, wp)
    s2 = _combine_relu_mm(p, W2, b2)
    q = _sc_scatter(s2, srcp, dstp, wp)
    return _combine_add(q)


# speedup vs baseline: 1.8171x; 1.0781x over previous
import functools

import jax
import jax.numpy as jnp
from jax import lax
from jax.experimental import pallas as pl
from jax.experimental.pallas import tpu as pltpu
from jax.experimental.pallas import tpu_sc as plsc

_N = 10000
_E = 320000
_D = 128

_NC = 2
_NS = 16
_NW = _NC * _NS
_CHUNK = 128
_NCHUNK = 79
_EPW = _NCHUNK * _CHUNK
_NPAD = 10240
_ROWS_PER_TILE = _NPAD // _NS
_ZROWS = _CHUNK


def _mm_bias_body(x_ref, w_ref, b_ref, o_ref):
    o_ref[...] = (
        jnp.dot(x_ref[...], w_ref[...], preferred_element_type=jnp.float32)
        + b_ref[...]
    )


def _mm_bias(x, w, b):
    return pl.pallas_call(
        _mm_bias_body,
        out_shape=jax.ShapeDtypeStruct((x.shape[0], w.shape[1]), jnp.float32),
    )(x, w, b.reshape(1, -1))


def _combine_relu_mm_body(p_ref, w_ref, b_ref, o_ref):
    h = jnp.maximum(p_ref[0, :_N, :] + p_ref[1, :_N, :], 0.0)
    o_ref[...] = (
        jnp.dot(h, w_ref[...], preferred_element_type=jnp.float32)
        + b_ref[...]
    )


def _combine_relu_mm(p, w, b):
    return pl.pallas_call(
        _combine_relu_mm_body,
        out_shape=jax.ShapeDtypeStruct((_N, w.shape[1]), jnp.float32),
    )(p, w, b.reshape(1, -1))


def _add_body(q_ref, o_ref):
    o_ref[...] = q_ref[0, :_N, :] + q_ref[1, :_N, :]


def _combine_add(q):
    return pl.pallas_call(
        _add_body,
        out_shape=jax.ShapeDtypeStruct((_N, _D), jnp.float32),
    )(q)


def _bcast_lane(vec16, k):
    idx = jnp.full((16, 1), k, jnp.int32)
    dnums = lax.GatherDimensionNumbers(
        offset_dims=(), collapsed_slice_dims=(0,), start_index_map=(0,))
    return lax.gather(
        vec16, idx, dnums, (1,),
        mode=lax.GatherScatterMode.PROMISE_IN_BOUNDS)


def _sc_body(sup_hbm, src_hbm, dst_hbm, w_hbm, out_hbm,
             src_v, dst_v, w_v, rows_v, acc, sem):
    c = lax.axis_index("c")
    s = lax.axis_index("s")
    wid = s * _NC + c

    pltpu.sync_copy(src_hbm.at[wid], src_v)
    pltpu.sync_copy(dst_hbm.at[wid], dst_v)
    pltpu.sync_copy(w_hbm.at[wid], w_v)

    def _zrow(r, carry):
        for cc in range(_D // 16):
            rows_v[r, pl.ds(cc * 16, 16)] = jnp.zeros((16,), jnp.float32)
        return carry
    lax.fori_loop(0, _ZROWS, _zrow, 0, unroll=4)
    base = s * _ROWS_PER_TILE
    for z in range(_ROWS_PER_TILE // _ZROWS):
        pltpu.sync_copy(
            rows_v,
            acc.at[pl.ds(base + z * _ZROWS, _ZROWS)],
        )
    plsc.subcore_barrier()

    def _chunk(j, carry):
        pltpu.async_copy(sup_hbm.at[src_v.at[j]], rows_v, sem).wait()

        def _grp(g, gcarry):
            wv = w_v[j, pl.ds(g * 16, 16)]
            for k in range(16):
                r = g * 16 + k
                wvec = _bcast_lane(wv, k)
                for cc in range(_D // 16):
                    sl = pl.ds(cc * 16, 16)
                    rows_v[r, sl] = rows_v[r, sl] * wvec
            return gcarry

        lax.fori_loop(0, _CHUNK // 16, _grp, 0)
        pltpu.sync_copy(rows_v, acc.at[dst_v.at[j]], add=True)
        return carry

    lax.fori_loop(0, _NCHUNK, _chunk, 0)
    plsc.subcore_barrier()

    pltpu.sync_copy(
        acc.at[pl.ds(base, _ROWS_PER_TILE)],
        out_hbm.at[c, pl.ds(base, _ROWS_PER_TILE)],
    )


_sc_scatter = functools.partial(
    pl.kernel,
    mesh=plsc.VectorSubcoreMesh(
        core_axis_name="c", subcore_axis_name="s",
        num_cores=_NC, num_subcores=_NS),
    out_type=jax.ShapeDtypeStruct((_NC, _NPAD, _D), jnp.float32),
    scratch_types=[
        pltpu.VMEM((_NCHUNK, _CHUNK), jnp.int32),
        pltpu.VMEM((_NCHUNK, _CHUNK), jnp.int32),
        pltpu.VMEM((_NCHUNK, _CHUNK), jnp.float32),
        pltpu.VMEM((_CHUNK, _D), jnp.float32),
        pltpu.VMEM_SHARED((_NPAD, _D), jnp.float32),
        pltpu.SemaphoreType.DMA,
    ],
)(_sc_body)


@jax.jit
def kernel(inp, edge_index, edge_weight, W1, b1, W2, b2):
    src = edge_index[0]
    dst = edge_index[1]
    pad = _NW * _EPW - _E
    srcp = jnp.concatenate(
        [src, jnp.zeros((pad,), jnp.int32)]).reshape(_NW, _NCHUNK, _CHUNK)
    dstp = jnp.concatenate(
        [dst, jnp.zeros((pad,), jnp.int32)]).reshape(_NW, _NCHUNK, _CHUNK)
    wp = jnp.concatenate(
        [edge_weight, jnp.zeros((pad,), jnp.float32)]
    ).reshape(_NW, _NCHUNK, _CHUNK)

    s1 = _mm_bias(inp, W1, b1)
    p = _sc_scatter(s1, srcp, dstp, wp)
    s2 = _combine_relu_mm(p, W2, b2)
    q = _sc_scatter(s2, srcp, dstp, wp)
    return _combine_add(q)
